# Initial kernel scaffold; baseline (speedup 1.0000x reference)
import jax
import jax.numpy as jnp
from jax.experimental import pallas as pl


def kernel(a, h_values, h_indices, g, action_mask, n_nodes, W_node, W_act, b_act):
    raise NotImplementedError("write your pallas kernel here")


# speedup vs baseline: 4.4655x; 4.4655x over previous
import functools

import jax
import jax.numpy as jnp
from jax import lax
from jax.experimental import pallas as pl
from jax.experimental.pallas import tpu as pltpu
from jax.experimental.pallas import tpu_sc as plsc

N = 100000
D = 128
B = 1024
A = 8
NEG = -1e9
CHUNK = 128
BLK_A = 2000
NW = 32
GPT = B // NW


def _stage_a_body(h_ref, mask_ref, w_ref, out_ref):
    h = h_ref[...]
    m = mask_ref[...]
    w = w_ref[...]
    logits = lax.dot_general(h, w, (((1,), (1,)), ((), ())),
                             preferred_element_type=jnp.float32)
    mf = m.astype(jnp.float32)
    nmask = jnp.sum(mf[:, 1:], axis=1, keepdims=True) > 0.0
    mlog = jnp.where(nmask, logits, NEG)
    out_ref[...] = jnp.concatenate([mlog, mf], axis=1)


def _stage_a(h_values, action_mask, w_node):
    nblk = N // BLK_A
    return pl.pallas_call(
        _stage_a_body,
        grid=(nblk,),
        in_specs=[
            pl.BlockSpec((BLK_A, D), lambda i: (i, 0)),
            pl.BlockSpec((BLK_A, A), lambda i: (i, 0)),
            pl.BlockSpec((A, D), lambda i: (0, 0)),
        ],
        out_specs=pl.BlockSpec((BLK_A, 16), lambda i: (i, 0)),
        out_shape=jax.ShapeDtypeStruct((N + CHUNK, 16), jnp.float32),
    )(h_values, action_mask, w_node)


def _lane_select(vec, k, zero):
    lane = lax.broadcasted_iota(jnp.int32, (16,), 0)
    return jnp.sum(jnp.where(lane == k, vec, zero))


def _stage_b_body(packed1d, nnodes,
                  mp_out, zs_out, ts_out,
                  nn_v, buf, st_m, st_z, st_t):
    cid = lax.axis_index("c")
    sid = lax.axis_index("s")
    wid = sid * 2 + cid
    base_g = pl.multiple_of(wid * GPT, GPT)

    pltpu.sync_copy(nnodes, nn_v)

    def _sum_body(j, acc):
        return acc + nn_v[pl.ds(j * 16, 16)]
    acc = lax.fori_loop(0, base_g // 16, _sum_body,
                        jnp.zeros((16,), jnp.int32))
    base_node = jnp.sum(acc)

    cnt0 = nn_v[pl.ds(base_g, 16)]
    cnt1 = nn_v[pl.ds(base_g + 16, 16)]

    neg16 = jnp.full((16,), NEG, jnp.float32)
    zero16 = jnp.zeros((16,), jnp.float32)

    def _row(j):
        return buf[pl.ds(pl.multiple_of(j * 16, 16), 16)]

    def _graph_body(i, start):
        cnt = _lane_select(jnp.where(i < 16, cnt0, cnt1), i % 16,
                           jnp.zeros((16,), jnp.int32))

        def _win(c):
            return pl.ds(pl.multiple_of((start + c * CHUNK) * 16, 16),
                         CHUNK * 16)

        def _pass1_chunk(c, m_acc):
            pltpu.sync_copy(packed1d.at[_win(c)], buf)
            valid = jnp.minimum(cnt - c * CHUNK, CHUNK)

            def _p1(j, m):
                return jnp.maximum(m, _row(j))
            return lax.fori_loop(0, valid, _p1, m_acc)

        nchunks = (cnt + CHUNK - 1) // CHUNK
        m_fin = lax.fori_loop(0, nchunks, _pass1_chunk, neg16)

        def _pass2_chunk(c, zt):
            pltpu.sync_copy(packed1d.at[_win(c)], buf)
            valid = jnp.minimum(cnt - c * CHUNK, CHUNK)

            def _p2(j, zt_in):
                z, t = zt_in
                s = _row(j) - m_fin
                e = jnp.exp(s)
                return (z + e, t + e * s)
            return lax.fori_loop(0, valid, _p2, zt)

        z_fin, t_fin = lax.fori_loop(0, nchunks, _pass2_chunk,
                                     (zero16, zero16))
        st_m[i, :] = m_fin
        st_z[i, :] = z_fin
        st_t[i, :] = t_fin
        return start + cnt

    lax.fori_loop(0, GPT, _graph_body, base_node)

    pltpu.sync_copy(st_m, mp_out.at[pl.ds(base_g, GPT)])
    pltpu.sync_copy(st_z, zs_out.at[pl.ds(base_g, GPT)])
    pltpu.sync_copy(st_t, ts_out.at[pl.ds(base_g, GPT)])


def _gather_body(packed2d, nodeids, hidx2,
                 rowvals_out, gb_out,
                 nid_v, rows_v, gb_v, sem):
    cid = lax.axis_index("c")
    sid = lax.axis_index("s")
    wid = sid * 2 + cid
    base_g = pl.multiple_of(wid * GPT, GPT)

    pltpu.sync_copy(nodeids.at[pl.ds(base_g, GPT)], nid_v)
    pltpu.async_copy(packed2d.at[nid_v], rows_v, sem).wait()
    pltpu.sync_copy(rows_v, rowvals_out.at[pl.ds(base_g, GPT)])
    pltpu.async_copy(hidx2.at[nid_v], gb_v, sem).wait()
    pltpu.sync_copy(gb_v, gb_out.at[pl.ds(base_g, GPT)])


def _sc_mesh():
    return plsc.VectorSubcoreMesh(core_axis_name="c", subcore_axis_name="s",
                                  num_cores=2, num_subcores=16)


def _stage_b(packed, n_nodes, node_ids, hidx2):
    f = pl.kernel(
        _stage_b_body,
        out_type=(
            jax.ShapeDtypeStruct((B, 16), jnp.float32),
            jax.ShapeDtypeStruct((B, 16), jnp.float32),
            jax.ShapeDtypeStruct((B, 16), jnp.float32),
        ),
        mesh=_sc_mesh(),
        compiler_params=pltpu.CompilerParams(use_tc_tiling_on_sc=False,
                                             needs_layout_passes=False),
        scratch_types=[
            pltpu.VMEM((B,), jnp.int32),
            pltpu.VMEM((CHUNK * 16,), jnp.float32),
            pltpu.VMEM((GPT, 16), jnp.float32),
            pltpu.VMEM((GPT, 16), jnp.float32),
            pltpu.VMEM((GPT, 16), jnp.float32),
        ],
    )
    mp, zs, ts = f(packed.reshape(-1), n_nodes)

    fg = pl.kernel(
        _gather_body,
        out_type=(
            jax.ShapeDtypeStruct((B, 16), jnp.float32),
            jax.ShapeDtypeStruct((B, 1), jnp.int32),
        ),
        mesh=_sc_mesh(),
        compiler_params=pltpu.CompilerParams(use_tc_tiling_on_sc=False,
                                             needs_layout_passes=False),
        scratch_types=[
            pltpu.VMEM((GPT,), jnp.int32),
            pltpu.VMEM((GPT, 16), jnp.float32),
            pltpu.VMEM((GPT, 1), jnp.int32),
            pltpu.SemaphoreType.DMA,
        ],
    )
    rowvals, gb = fg(packed, node_ids, hidx2)
    return mp, zs, ts, rowvals, gb


def _stage_c_body(mp_ref, zs_ref, ts_ref, rv_ref, gb_ref, act_ref,
                  g_ref, wact_ref, bact_ref, lp_ref, ent_ref):
    mp = mp_ref[...]
    m_seg = mp[:, :A]
    pred = mp[:, A:] > 0.5
    z_seg = zs_ref[...][:, :A]
    t_seg = ts_ref[...][:, :A]
    gvec = g_ref[...]
    wact = wact_ref[...]
    bact = bact_ref[...]
    act = act_ref[...]
    gb = gb_ref[...]
    rowv = rv_ref[...][:, :A]

    action_logits = lax.dot_general(gvec, wact, (((1,), (1,)), ((), ())),
                                    preferred_element_type=jnp.float32) + bact
    a_logits = jnp.where(pred, action_logits, NEG)
    amax = jnp.max(a_logits, axis=1, keepdims=True)
    sh = a_logits - amax
    lse = jnp.log(jnp.sum(jnp.exp(sh), axis=1, keepdims=True))
    logp_a = sh - lse
    p_a = jnp.exp(logp_a)
    h_a = -jnp.sum(p_a * jnp.where(pred, logp_a, 0.0), axis=1)

    zc = jnp.maximum(z_seg, 1e-30)
    log_z = jnp.log(zc)
    any_nm = jnp.sum(pred[:, 1:].astype(jnp.float32), axis=1,
                     keepdims=True) > 0.0
    h_n = jnp.where(any_nm, log_z - t_seg / zc, 0.0)
    entropy = h_a + jnp.sum(p_a[:, 1:] * h_n[:, 1:], axis=1)

    iota_b = lax.broadcasted_iota(jnp.int32, (B, B), 1)
    oh_gb = (iota_b == gb).astype(jnp.float32)
    stats = jnp.concatenate([m_seg, log_z], axis=1)
    stats_g = lax.dot_general(oh_gb, stats, (((1,), (0,)), ((), ())),
                              preferred_element_type=jnp.float32)

    iota_a = lax.broadcasted_iota(jnp.int32, (B, A), 1)
    sel = (iota_a == act).astype(jnp.float32)
    lp_act = jnp.sum(sel * logp_a, axis=1)
    mval = jnp.sum(sel * rowv, axis=1)
    m_gb = jnp.sum(sel * stats_g[:, :A], axis=1)
    lz_gb = jnp.sum(sel * stats_g[:, A:], axis=1)
    lp_node = mval - m_gb - lz_gb
    needs = (act[:, 0] != 0).astype(jnp.float32)
    lp_ref[...] = lp_act + needs * lp_node
    ent_ref[...] = entropy


def _stage_c(mp, zs, ts, rowvals, gb, act, g, w_act, b_act):
    return pl.pallas_call(
        _stage_c_body,
        out_shape=(jax.ShapeDtypeStruct((B,), jnp.float32),
                   jax.ShapeDtypeStruct((B,), jnp.float32)),
    )(mp, zs, ts, rowvals, gb, act, g, w_act, b_act)


@jax.jit
def kernel(a, h_values, h_indices, g, action_mask, n_nodes, W_node, W_act, b_act):
    packed = _stage_a(h_values, action_mask, W_node)
    node_ids = a[:, 1].astype(jnp.int32)
    hidx2 = h_indices.astype(jnp.int32).reshape(N, 1)
    mp, zs, ts, rowvals, gb = _stage_b(packed, n_nodes.astype(jnp.int32),
                                       node_ids, hidx2)
    act = a[:, 0:1].astype(jnp.int32)
    b_act2 = b_act.reshape(1, A)
    return _stage_c(mp, zs, ts, rowvals, gb, act, g, W_act, b_act2)


# speedup vs baseline: 5.2548x; 1.1767x over previous
import functools

import jax
import jax.numpy as jnp
from jax import lax
from jax.experimental import pallas as pl
from jax.experimental.pallas import tpu as pltpu
from jax.experimental.pallas import tpu_sc as plsc

N = 100000
D = 128
B = 1024
A = 8
NEG = -1e9
CHUNK = 128
BIGBUF = 6144
DCH = 512
BLK_A = 2000
NW = 32
GPT = B // NW


def _stage_a_body(h_ref, mask_ref, w_ref, out_ref):
    h = h_ref[...]
    m = mask_ref[...]
    w = w_ref[...]
    logits = lax.dot_general(h, w, (((1,), (1,)), ((), ())),
                             preferred_element_type=jnp.float32)
    mf = m.astype(jnp.float32)
    nmask = jnp.sum(mf[:, 1:], axis=1, keepdims=True) > 0.0
    mlog = jnp.where(nmask, logits, NEG)
    out_ref[...] = jnp.concatenate([mlog, mf], axis=1)


def _stage_a(h_values, action_mask, w_node):
    nblk = N // BLK_A
    return pl.pallas_call(
        _stage_a_body,
        grid=(nblk,),
        in_specs=[
            pl.BlockSpec((BLK_A, D), lambda i: (i, 0)),
            pl.BlockSpec((BLK_A, A), lambda i: (i, 0)),
            pl.BlockSpec((A, D), lambda i: (0, 0)),
        ],
        out_specs=pl.BlockSpec((BLK_A, 16), lambda i: (i, 0)),
        out_shape=jax.ShapeDtypeStruct((N + DCH, 16), jnp.float32),
    )(h_values, action_mask, w_node)


def _lane_select(vec, k, zero):
    lane = lax.broadcasted_iota(jnp.int32, (16,), 0)
    return jnp.sum(jnp.where(lane == k, vec, zero))


def _stage_b_body(packed1d, nnodes,
                  mp_out, zs_out, ts_out,
                  nn_v, bigbuf, st_m, st_z, st_t):
    cid = lax.axis_index("c")
    sid = lax.axis_index("s")
    wid = sid * 2 + cid
    base_g = pl.multiple_of(wid * GPT, GPT)

    pltpu.sync_copy(nnodes, nn_v)

    def _sum_body(j, acc):
        return acc + nn_v[pl.ds(j * 16, 16)]
    acc = lax.fori_loop(0, base_g // 16, _sum_body,
                        jnp.zeros((16,), jnp.int32))
    base_node = jnp.sum(acc)

    cnt0 = nn_v[pl.ds(base_g, 16)]
    cnt1 = nn_v[pl.ds(base_g + 16, 16)]
    tcnt = jnp.sum(cnt0) + jnp.sum(cnt1)

    neg16 = jnp.full((16,), NEG, jnp.float32)
    zero16 = jnp.zeros((16,), jnp.float32)
    izero16 = jnp.zeros((16,), jnp.int32)

    def _rowv(r):
        return bigbuf[pl.ds(pl.multiple_of(r * 16, 16), 16)]

    def _cnt_of(i):
        return _lane_select(jnp.where(i < 16, cnt0, cnt1), i % 16, izero16)

    def _reduce_graph(i, rel, cnt):
        q4 = cnt // 4

        def _p1(j, ms):
            r = rel + j * 4
            m0, m1, m2, m3 = ms
            return (jnp.maximum(m0, _rowv(r)), jnp.maximum(m1, _rowv(r + 1)),
                    jnp.maximum(m2, _rowv(r + 2)), jnp.maximum(m3, _rowv(r + 3)))
        m0, m1, m2, m3 = lax.fori_loop(0, q4, _p1, (neg16, neg16, neg16, neg16))
        m = jnp.maximum(jnp.maximum(m0, m1), jnp.maximum(m2, m3))

        def _p1r(j, mm):
            return jnp.maximum(mm, _rowv(rel + q4 * 4 + j))
        m = lax.fori_loop(0, cnt - q4 * 4, _p1r, m)

        def _p2(j, zt):
            r = rel + j * 4
            z0, z1, z2, z3, t0, t1, t2, t3 = zt
            s0 = _rowv(r) - m
            s1 = _rowv(r + 1) - m
            s2 = _rowv(r + 2) - m
            s3 = _rowv(r + 3) - m
            e0 = jnp.exp(s0)
            e1 = jnp.exp(s1)
            e2 = jnp.exp(s2)
            e3 = jnp.exp(s3)
            return (z0 + e0, z1 + e1, z2 + e2, z3 + e3,
                    t0 + e0 * s0, t1 + e1 * s1, t2 + e2 * s2, t3 + e3 * s3)
        z0, z1, z2, z3, t0, t1, t2, t3 = lax.fori_loop(
            0, q4, _p2, (zero16,) * 8)
        z = (z0 + z1) + (z2 + z3)
        t = (t0 + t1) + (t2 + t3)

        def _p2r(j, zt):
            zz, tt = zt
            s = _rowv(rel + q4 * 4 + j) - m
            e = jnp.exp(s)
            return (zz + e, tt + e * s)
        z, t = lax.fori_loop(0, cnt - q4 * 4, _p2r, (z, t))

        st_m[i, :] = m
        st_z[i, :] = z
        st_t[i, :] = t

    @pl.when(tcnt <= BIGBUF)
    def _fast():
        nch = (tcnt + DCH - 1) // DCH

        def _dma(c, _):
            pltpu.sync_copy(
                packed1d.at[pl.ds(
                    pl.multiple_of((base_node + c * DCH) * 16, 16), DCH * 16)],
                bigbuf.at[pl.ds(pl.multiple_of(c * DCH * 16, 16), DCH * 16)])
            return 0
        lax.fori_loop(0, nch, _dma, 0)

        def _graph(i, rel):
            cnt = _cnt_of(i)
            _reduce_graph(i, rel, cnt)
            return rel + cnt
        lax.fori_loop(0, GPT, _graph, jnp.int32(0))

    @pl.when(tcnt > BIGBUF)
    def _slow():
        def _graph_body(i, start):
            cnt = _cnt_of(i)

            def _win(c):
                return pl.ds(pl.multiple_of((start + c * CHUNK) * 16, 16),
                             CHUNK * 16)

            def _pass1_chunk(c, m_acc):
                pltpu.sync_copy(
                    packed1d.at[_win(c)],
                    bigbuf.at[pl.ds(0, CHUNK * 16)])
                valid = jnp.minimum(cnt - c * CHUNK, CHUNK)

                def _p1(j, mm):
                    return jnp.maximum(mm, _rowv(j))
                return lax.fori_loop(0, valid, _p1, m_acc)

            nchunks = (cnt + CHUNK - 1) // CHUNK
            m_fin = lax.fori_loop(0, nchunks, _pass1_chunk, neg16)

            def _pass2_chunk(c, zt):
                pltpu.sync_copy(
                    packed1d.at[_win(c)],
                    bigbuf.at[pl.ds(0, CHUNK * 16)])
                valid = jnp.minimum(cnt - c * CHUNK, CHUNK)

                def _p2(j, zt_in):
                    z, t = zt_in
                    s = _rowv(j) - m_fin
                    e = jnp.exp(s)
                    return (z + e, t + e * s)
                return lax.fori_loop(0, valid, _p2, zt)

            z_fin, t_fin = lax.fori_loop(0, nchunks, _pass2_chunk,
                                         (zero16, zero16))
            st_m[i, :] = m_fin
            st_z[i, :] = z_fin
            st_t[i, :] = t_fin
            return start + cnt

        lax.fori_loop(0, GPT, _graph_body, base_node)

    pltpu.sync_copy(st_m, mp_out.at[pl.ds(base_g, GPT)])
    pltpu.sync_copy(st_z, zs_out.at[pl.ds(base_g, GPT)])
    pltpu.sync_copy(st_t, ts_out.at[pl.ds(base_g, GPT)])


def _gather_body(packed2d, nodeids, hidx2,
                 rowvals_out, gb_out,
                 nid_v, rows_v, gb_v, sem):
    cid = lax.axis_index("c")
    sid = lax.axis_index("s")
    wid = sid * 2 + cid
    base_g = pl.multiple_of(wid * GPT, GPT)

    pltpu.sync_copy(nodeids.at[pl.ds(base_g, GPT)], nid_v)
    pltpu.async_copy(packed2d.at[nid_v], rows_v, sem).wait()
    pltpu.sync_copy(rows_v, rowvals_out.at[pl.ds(base_g, GPT)])
    pltpu.async_copy(hidx2.at[nid_v], gb_v, sem).wait()
    pltpu.sync_copy(gb_v, gb_out.at[pl.ds(base_g, GPT)])


def _sc_mesh():
    return plsc.VectorSubcoreMesh(core_axis_name="c", subcore_axis_name="s",
                                  num_cores=2, num_subcores=16)


def _stage_b(packed, n_nodes, node_ids, hidx2):
    f = pl.kernel(
        _stage_b_body,
        out_type=(
            jax.ShapeDtypeStruct((B, 16), jnp.float32),
            jax.ShapeDtypeStruct((B, 16), jnp.float32),
            jax.ShapeDtypeStruct((B, 16), jnp.float32),
        ),
        mesh=_sc_mesh(),
        compiler_params=pltpu.CompilerParams(use_tc_tiling_on_sc=False,
                                             needs_layout_passes=False),
        scratch_types=[
            pltpu.VMEM((B,), jnp.int32),
            pltpu.VMEM((BIGBUF * 16,), jnp.float32),
            pltpu.VMEM((GPT, 16), jnp.float32),
            pltpu.VMEM((GPT, 16), jnp.float32),
            pltpu.VMEM((GPT, 16), jnp.float32),
        ],
    )
    mp, zs, ts = f(packed.reshape(-1), n_nodes)

    fg = pl.kernel(
        _gather_body,
        out_type=(
            jax.ShapeDtypeStruct((B, 16), jnp.float32),
            jax.ShapeDtypeStruct((B, 1), jnp.int32),
        ),
        mesh=_sc_mesh(),
        compiler_params=pltpu.CompilerParams(use_tc_tiling_on_sc=False,
                                             needs_layout_passes=False),
        scratch_types=[
            pltpu.VMEM((GPT,), jnp.int32),
            pltpu.VMEM((GPT, 16), jnp.float32),
            pltpu.VMEM((GPT, 1), jnp.int32),
            pltpu.SemaphoreType.DMA,
        ],
    )
    rowvals, gb = fg(packed, node_ids, hidx2)
    return mp, zs, ts, rowvals, gb


def _stage_c_body(mp_ref, zs_ref, ts_ref, rv_ref, gb_ref, act_ref,
                  g_ref, wact_ref, bact_ref, lp_ref, ent_ref):
    mp = mp_ref[...]
    m_seg = mp[:, :A]
    pred = mp[:, A:] > 0.5
    z_seg = zs_ref[...][:, :A]
    t_seg = ts_ref[...][:, :A]
    gvec = g_ref[...]
    wact = wact_ref[...]
    bact = bact_ref[...]
    act = act_ref[...]
    gb = gb_ref[...]
    rowv = rv_ref[...][:, :A]

    action_logits = lax.dot_general(gvec, wact, (((1,), (1,)), ((), ())),
                                    preferred_element_type=jnp.float32) + bact
    a_logits = jnp.where(pred, action_logits, NEG)
    amax = jnp.max(a_logits, axis=1, keepdims=True)
    sh = a_logits - amax
    lse = jnp.log(jnp.sum(jnp.exp(sh), axis=1, keepdims=True))
    logp_a = sh - lse
    p_a = jnp.exp(logp_a)
    h_a = -jnp.sum(p_a * jnp.where(pred, logp_a, 0.0), axis=1)

    zc = jnp.maximum(z_seg, 1e-30)
    log_z = jnp.log(zc)
    any_nm = jnp.sum(pred[:, 1:].astype(jnp.float32), axis=1,
                     keepdims=True) > 0.0
    h_n = jnp.where(any_nm, log_z - t_seg / zc, 0.0)
    entropy = h_a + jnp.sum(p_a[:, 1:] * h_n[:, 1:], axis=1)

    iota_b = lax.broadcasted_iota(jnp.int32, (B, B), 1)
    oh_gb = (iota_b == gb).astype(jnp.float32)
    stats = jnp.concatenate([m_seg, log_z], axis=1)
    stats_g = lax.dot_general(oh_gb, stats, (((1,), (0,)), ((), ())),
                              preferred_element_type=jnp.float32)

    iota_a = lax.broadcasted_iota(jnp.int32, (B, A), 1)
    sel = (iota_a == act).astype(jnp.float32)
    lp_act = jnp.sum(sel * logp_a, axis=1)
    mval = jnp.sum(sel * rowv, axis=1)
    m_gb = jnp.sum(sel * stats_g[:, :A], axis=1)
    lz_gb = jnp.sum(sel * stats_g[:, A:], axis=1)
    lp_node = mval - m_gb - lz_gb
    needs = (act[:, 0] != 0).astype(jnp.float32)
    lp_ref[...] = lp_act + needs * lp_node
    ent_ref[...] = entropy


def _stage_c(mp, zs, ts, rowvals, gb, act, g, w_act, b_act):
    return pl.pallas_call(
        _stage_c_body,
        out_shape=(jax.ShapeDtypeStruct((B,), jnp.float32),
                   jax.ShapeDtypeStruct((B,), jnp.float32)),
    )(mp, zs, ts, rowvals, gb, act, g, w_act, b_act)


@jax.jit
def kernel(a, h_values, h_indices, g, action_mask, n_nodes, W_node, W_act, b_act):
    packed = _stage_a(h_values, action_mask, W_node)
    node_ids = a[:, 1].astype(jnp.int32)
    hidx2 = h_indices.astype(jnp.int32).reshape(N, 1)
    mp, zs, ts, rowvals, gb = _stage_b(packed, n_nodes.astype(jnp.int32),
                                       node_ids, hidx2)
    act = a[:, 0:1].astype(jnp.int32)
    b_act2 = b_act.reshape(1, A)
    return _stage_c(mp, zs, ts, rowvals, gb, act, g, W_act, b_act2)


# speedup vs baseline: 14.5654x; 2.7718x over previous
import jax
import jax.numpy as jnp
from jax import lax
from jax.experimental import pallas as pl
from jax.experimental.pallas import tpu as pltpu
from jax.experimental.pallas import tpu_sc as plsc

N = 100000
D = 128
B = 1024
A = 8
NEG = -1e9
BLK_A = 6656
NPAD = 106496
NROW = NPAD // 128
NW = 32
GPT = B // NW
SEGR = 40
SEGROW = SEGR * 128
SEGW = SEGROW - 128


def _stage_a_body(h_ref, maskt_ref, w_ref, out_ref):
    h = h_ref[...]
    mt = maskt_ref[...]
    w = w_ref[...]
    logits = lax.dot_general(h, w, (((1,), (1,)), ((), ())),
                             preferred_element_type=jnp.float32)
    logits_t = jnp.transpose(logits)
    mtf = mt.astype(jnp.float32)
    nmask = jnp.sum(mtf[1:], axis=0, keepdims=True) > 0.0
    mlog_t = jnp.where(nmask, logits_t, NEG)
    out_ref[...] = jnp.concatenate([mlog_t, mtf], axis=0)


def _stage_a(h_values, action_mask_t, w_node):
    nblk = NPAD // BLK_A
    return pl.pallas_call(
        _stage_a_body,
        grid=(nblk,),
        in_specs=[
            pl.BlockSpec((BLK_A, D), lambda i: (i, 0)),
            pl.BlockSpec((A, BLK_A), lambda i: (0, i)),
            pl.BlockSpec((A, D), lambda i: (0, 0)),
        ],
        out_specs=pl.BlockSpec((16, BLK_A), lambda i: (0, i)),
        out_shape=jax.ShapeDtypeStruct((16, NPAD), jnp.float32),
    )(h_values, action_mask_t, w_node)


def _lane_select(vec, k, zero):
    lane = lax.broadcasted_iota(jnp.int32, (16,), 0)
    return jnp.sum(jnp.where(lane == k, vec, zero))


def _stage_b_body(packed128, nnodes, nodeids,
                  mp_out, zs_out, ts_out, rowflat_out,
                  nn_v, bigbuf, st_m, st_z, st_t,
                  nid_v, gbuf, rows_flat, sem):
    cid = lax.axis_index("c")
    sid = lax.axis_index("s")
    wid = sid * 2 + cid
    base_g = pl.multiple_of(wid * GPT, GPT)

    pltpu.sync_copy(nnodes, nn_v)

    def _sum_body(j, acc):
        return acc + nn_v[pl.ds(j * 16, 16)]
    acc = lax.fori_loop(0, base_g // 16, _sum_body,
                        jnp.zeros((16,), jnp.int32))
    base_node = jnp.sum(acc)

    cnt0 = nn_v[pl.ds(base_g, 16)]
    cnt1 = nn_v[pl.ds(base_g + 16, 16)]
    tcnt = jnp.sum(cnt0) + jnp.sum(cnt1)

    lane = lax.broadcasted_iota(jnp.int32, (16,), 0)
    neg16 = jnp.full((16,), NEG, jnp.float32)
    zero16 = jnp.zeros((16,), jnp.float32)
    izero16 = jnp.zeros((16,), jnp.int32)

    def _cnt_of(i):
        return _lane_select(jnp.where(i < 16, cnt0, cnt1), i % 16, izero16)

    ts0 = pl.multiple_of((base_node // 128) * 128, 128)
    nseg = (base_node - ts0 + tcnt + SEGW - 1) // SEGW

    def _init(i, _):
        st_m[i, :] = jnp.where(lane < 8, neg16, zero16)
        st_z[i, :] = zero16
        st_t[i, :] = zero16
        return 0
    lax.fori_loop(0, GPT, _init, 0)

    def _dma_segment(w0):
        r0 = w0 // 128
        for a in range(16):
            pltpu.sync_copy(
                packed128.at[pl.ds(a * NROW + r0, SEGR), :],
                bigbuf.at[pl.ds(a * SEGR, SEGR), :])

    def _clip(gs, ge, w0):
        lo = jnp.maximum(gs, w0)
        hi = jnp.minimum(ge, w0 + SEGW)
        al = (lo // 16) * 16
        nv = jnp.where(hi > lo, (hi - al + 15) // 16, 0)
        return al, lo, hi, nv

    def _buf_vec(a, r0v, c0v):
        return bigbuf[a * SEGR + r0v, pl.ds(pl.multiple_of(c0v, 16), 16)]

    def _sweep1_seg(s, _):
        w0 = ts0 + s * SEGW
        _dma_segment(w0)

        def _graph(i, gs):
            cnt = _cnt_of(i)
            al, lo, hi, nv = _clip(gs, gs + cnt, w0)
            boff = al - w0

            def _scan(v, accs):
                pos = al + v * 16 + lane
                valid = (pos >= lo) & (pos < hi)
                w = boff + v * 16
                r0v = w // 128
                c0v = w % 128
                new = []
                for a in range(16):
                    vec = _buf_vec(a, r0v, c0v)
                    if a < 8:
                        new.append(jnp.maximum(
                            accs[a], jnp.where(valid, vec, NEG)))
                    else:
                        new.append(accs[a] + jnp.where(valid, vec, 0.0))
                return tuple(new)

            accs = lax.fori_loop(0, nv, _scan,
                                 (neg16,) * 8 + (zero16,) * 8)
            mrow = zero16
            prow = zero16
            for a in range(8):
                mrow = jnp.where(lane == a, jnp.max(accs[a]), mrow)
                prow = jnp.where(lane == (a + 8), jnp.sum(accs[a + 8]), prow)
            old = st_m[i, :]
            st_m[i, :] = jnp.where(lane < 8,
                                   jnp.maximum(old, mrow), old + prow)
            return gs + cnt
        lax.fori_loop(0, GPT, _graph, base_node)
        return 0

    lax.fori_loop(0, nseg, _sweep1_seg, 0)

    def _sweep2_seg(s, _):
        w0 = ts0 + s * SEGW

        @pl.when(nseg > 1)
        def _():
            _dma_segment(w0)

        def _graph(i, gs):
            cnt = _cnt_of(i)
            al, lo, hi, nv = _clip(gs, gs + cnt, w0)
            boff = al - w0
            mrow = st_m[i, :]
            mss = [_lane_select(mrow, a, zero16) for a in range(8)]

            def _scan(v, accs):
                pos = al + v * 16 + lane
                valid = (pos >= lo) & (pos < hi)
                w = boff + v * 16
                r0v = w // 128
                c0v = w % 128
                new = list(accs)
                for a in range(8):
                    vec = _buf_vec(a, r0v, c0v)
                    sh = jnp.where(valid, vec - mss[a], 0.0)
                    e = jnp.where(valid, jnp.exp(sh), 0.0)
                    new[a] = accs[a] + e
                    new[a + 8] = accs[a + 8] + e * sh
                return tuple(new)

            accs = lax.fori_loop(0, nv, _scan, (zero16,) * 16)
            zrow = zero16
            trow = zero16
            for a in range(8):
                zrow = jnp.where(lane == a, jnp.sum(accs[a]), zrow)
                trow = jnp.where(lane == a, jnp.sum(accs[a + 8]), trow)
            st_z[i, :] = st_z[i, :] + zrow
            st_t[i, :] = st_t[i, :] + trow
            return gs + cnt
        lax.fori_loop(0, GPT, _graph, base_node)
        return 0

    lax.fori_loop(0, nseg, _sweep2_seg, 0)

    pltpu.sync_copy(st_m, mp_out.at[pl.ds(base_g, GPT)])
    pltpu.sync_copy(st_z, zs_out.at[pl.ds(base_g, GPT)])
    pltpu.sync_copy(st_t, ts_out.at[pl.ds(base_g, GPT)])

    pltpu.sync_copy(nodeids.at[pl.ds(base_g, GPT)], nid_v)
    nidv0 = nid_v[pl.ds(0, 16)]
    nidv1 = nid_v[pl.ds(16, 16)]
    for half in range(2):
        nids = []
        gcopies = []
        for jj in range(16):
            j = half * 16 + jj
            src = nidv0 if half == 0 else nidv1
            nid = jnp.sum(jnp.where(lane == jj, src, izero16))
            nids.append(nid)
            ridx = lane * NROW + nid // 128
            gcopies.append(pltpu.async_copy(
                packed128.at[ridx], gbuf.at[pl.ds(jj * 16, 16), :], sem))
        for c in gcopies:
            c.wait()
        for jj in range(16):
            j = half * 16 + jj
            col = jnp.full((16,), nids[jj] % 128, jnp.int32)
            vals = plsc.load_gather(gbuf, [jj * 16 + lane, col])
            rows_flat[pl.ds(j * 16, 16)] = vals
    pltpu.sync_copy(rows_flat, rowflat_out.at[pl.ds(base_g * 16, GPT * 16)])


def _stage_b(packed128, n_nodes, node_ids):
    mesh = plsc.VectorSubcoreMesh(core_axis_name="c", subcore_axis_name="s",
                                  num_cores=2, num_subcores=16)
    f = pl.kernel(
        _stage_b_body,
        out_type=(
            jax.ShapeDtypeStruct((B, 16), jnp.float32),
            jax.ShapeDtypeStruct((B, 16), jnp.float32),
            jax.ShapeDtypeStruct((B, 16), jnp.float32),
            jax.ShapeDtypeStruct((B * 16,), jnp.float32),
        ),
        mesh=mesh,
        compiler_params=pltpu.CompilerParams(use_tc_tiling_on_sc=False,
                                             needs_layout_passes=False),
        scratch_types=[
            pltpu.VMEM((B,), jnp.int32),
            pltpu.VMEM((16 * SEGR, 128), jnp.float32),
            pltpu.VMEM((GPT, 16), jnp.float32),
            pltpu.VMEM((GPT, 16), jnp.float32),
            pltpu.VMEM((GPT, 16), jnp.float32),
            pltpu.VMEM((GPT,), jnp.int32),
            pltpu.VMEM((256, 128), jnp.float32),
            pltpu.VMEM((GPT * 16,), jnp.float32),
            pltpu.SemaphoreType.DMA,
        ],
    )
    mp, zs, ts, rowflat = f(packed128, n_nodes, node_ids)
    return mp, zs, ts, rowflat.reshape(B, 16)


def _stage_c_body(mp_ref, zs_ref, ts_ref, rv_ref, node_ref, act_ref,
                  nn_ref, g_ref, wact_ref, bact_ref, lp_ref, ent_ref):
    mp = mp_ref[...]
    m_seg = mp[:, :A]
    pred = mp[:, A:] > 0.5
    z_seg = zs_ref[...][:, :A]
    t_seg = ts_ref[...][:, :A]
    gvec = g_ref[...]
    wact = wact_ref[...]
    bact = bact_ref[...]
    act = act_ref[...]
    node = node_ref[...]
    nnrow = nn_ref[...]
    rowv = rv_ref[...][:, :A]

    iota_r = lax.broadcasted_iota(jnp.int32, (B, B), 0)
    iota_c = lax.broadcasted_iota(jnp.int32, (B, B), 1)
    ltm = (iota_r < iota_c).astype(jnp.float32)
    starts = lax.dot_general(nnrow, ltm, (((1,), (0,)), ((), ())),
                             preferred_element_type=jnp.float32)
    cmp = (starts <= node.astype(jnp.float32)).astype(jnp.float32)
    gb = jnp.sum(cmp, axis=1, keepdims=True) - 1.0

    action_logits = lax.dot_general(gvec, wact, (((1,), (1,)), ((), ())),
                                    preferred_element_type=jnp.float32) + bact
    a_logits = jnp.where(pred, action_logits, NEG)
    amax = jnp.max(a_logits, axis=1, keepdims=True)
    sh = a_logits - amax
    lse = jnp.log(jnp.sum(jnp.exp(sh), axis=1, keepdims=True))
    logp_a = sh - lse
    p_a = jnp.exp(logp_a)
    h_a = -jnp.sum(p_a * jnp.where(pred, logp_a, 0.0), axis=1)

    zc = jnp.maximum(z_seg, 1e-30)
    log_z = jnp.log(zc)
    any_nm = jnp.sum(pred[:, 1:].astype(jnp.float32), axis=1,
                     keepdims=True) > 0.0
    h_n = jnp.where(any_nm, log_z - t_seg / zc, 0.0)
    entropy = h_a + jnp.sum(p_a[:, 1:] * h_n[:, 1:], axis=1)

    oh_gb = (iota_c.astype(jnp.float32) == gb).astype(jnp.float32)
    stats = jnp.concatenate([m_seg, log_z], axis=1)
    stats_g = lax.dot_general(oh_gb, stats, (((1,), (0,)), ((), ())),
                              preferred_element_type=jnp.float32)

    iota_a = lax.broadcasted_iota(jnp.int32, (B, A), 1)
    sel = (iota_a == act).astype(jnp.float32)
    lp_act = jnp.sum(sel * logp_a, axis=1)
    mval = jnp.sum(sel * rowv, axis=1)
    m_gb = jnp.sum(sel * stats_g[:, :A], axis=1)
    lz_gb = jnp.sum(sel * stats_g[:, A:], axis=1)
    lp_node = mval - m_gb - lz_gb
    needs = (act[:, 0] != 0).astype(jnp.float32)
    lp_ref[...] = lp_act + needs * lp_node
    ent_ref[...] = entropy


def _stage_c(mp, zs, ts, rowvals, node2, act, nn2, g, w_act, b_act):
    return pl.pallas_call(
        _stage_c_body,
        out_shape=(jax.ShapeDtypeStruct((B,), jnp.float32),
                   jax.ShapeDtypeStruct((B,), jnp.float32)),
    )(mp, zs, ts, rowvals, node2, act, nn2, g, w_act, b_act)


@jax.jit
def kernel(a, h_values, h_indices, g, action_mask, n_nodes, W_node, W_act, b_act):
    del h_indices
    packed = _stage_a(h_values, action_mask.T, W_node)
    packed128 = packed.reshape(16 * NPAD // 128, 128)
    node_ids = a[:, 1].astype(jnp.int32)
    nn32 = n_nodes.astype(jnp.int32)
    mp, zs, ts, rowvals = _stage_b(packed128, nn32, node_ids)
    act = a[:, 0:1].astype(jnp.int32)
    node2 = a[:, 1:2].astype(jnp.int32)
    nn2 = nn32.astype(jnp.float32).reshape(1, B)
    b_act2 = b_act.reshape(1, A)
    return _stage_c(mp, zs, ts, rowvals, node2, act, nn2, g, W_act, b_act2)
